# Initial kernel scaffold; baseline (speedup 1.0000x reference)
import jax
import jax.numpy as jnp
from jax.experimental import pallas as pl


def kernel(query, keys, values):
    raise NotImplementedError("write your pallas kernel here")


# speedup vs baseline: 1.6357x; 1.6357x over previous
import functools

import jax
import jax.numpy as jnp
from jax.experimental import pallas as pl
from jax.experimental.pallas import tpu as pltpu

B = 1024
M = 100000
KD = 16
VD = 16
M_BLK = 2000
NB = M // M_BLK


def _flash_body(q_ref, k_ref, v_ref, o_ref, m_ref, l_ref, acc_ref):
    i = pl.program_id(0)

    @pl.when(i == 0)
    def _init():
        m_ref[...] = jnp.full_like(m_ref, -1e30)
        l_ref[...] = jnp.zeros_like(l_ref)
        acc_ref[...] = jnp.zeros_like(acc_ref)

    q = q_ref[...]
    k = k_ref[...]
    s = jax.lax.dot_general(
        q, k, (((1,), (1,)), ((), ())), preferred_element_type=jnp.float32
    )
    m_prev = m_ref[...]
    m_new = jnp.maximum(m_prev, jnp.max(s, axis=1, keepdims=True))
    alpha = jnp.exp(m_prev - m_new)
    p = jnp.exp(s - m_new)
    m_ref[...] = m_new
    l_ref[...] = l_ref[...] * alpha + jnp.sum(p, axis=1, keepdims=True)
    acc_ref[...] = acc_ref[...] * alpha + jnp.dot(
        p, v_ref[...], preferred_element_type=jnp.float32
    )

    @pl.when(i == NB - 1)
    def _finish():
        o_ref[...] = acc_ref[...] / l_ref[...]


@jax.jit
def kernel(query, keys, values):
    return pl.pallas_call(
        _flash_body,
        grid=(NB,),
        in_specs=[
            pl.BlockSpec((B, KD), lambda i: (0, 0)),
            pl.BlockSpec((M_BLK, KD), lambda i: (i, 0)),
            pl.BlockSpec((M_BLK, VD), lambda i: (i, 0)),
        ],
        out_specs=pl.BlockSpec((B, VD), lambda i: (0, 0)),
        out_shape=jax.ShapeDtypeStruct((B, VD), jnp.float32),
        scratch_shapes=[
            pltpu.VMEM((B, 1), jnp.float32),
            pltpu.VMEM((B, 1), jnp.float32),
            pltpu.VMEM((B, VD), jnp.float32),
        ],
    )(query, keys, values)
